# baseline (speedup 1.0000x reference)
import functools

import jax
import jax.numpy as jnp
from jax import lax
from jax.experimental import pallas as pl
from jax.experimental.pallas import tpu as pltpu
from jax.experimental.pallas import tpu_sc as plsc

HIDDEN = 2048
NUM_TOKENS = 4096
NUM_ROWS = 16384

NUM_CORES = 2
NUM_SUBCORES = 16
NUM_WORKERS = NUM_CORES * NUM_SUBCORES
ROWS_PER_WORKER = NUM_ROWS // NUM_WORKERS
W = 16
CROWS = 24
_SIZES = [CROWS] * (ROWS_PER_WORKER // CROWS) + (
    [ROWS_PER_WORKER % CROWS] if ROWS_PER_WORKER % CROWS else [])
_OFFS = [sum(_SIZES[:i]) for i in range(len(_SIZES))]
NCH = len(_SIZES)


def _body(text_hbm, image_hbm, idx_hbm, src_hbm, lo_hbm, out_hbm,
          lov_v, widx_v, wsrc_v, cbuf_a, cbuf_b, rsem, wsem,
          gsem, ssem, isem):
    sid = lax.axis_index("s")
    wid = sid * NUM_CORES + lax.axis_index("c")
    r0 = wid * ROWS_PER_WORKER

    pltpu.sync_copy(lo_hbm.at[wid], lov_v)
    lvec = lov_v[...]
    seg_lo, seg_hi = lvec[0], lvec[1]
    lo8 = seg_lo & jnp.int32(~7)
    nwin = lax.select(
        seg_hi > seg_lo,
        lax.div(seg_hi - lo8 + jnp.int32(W - 1), jnp.int32(W)),
        jnp.int32(0),
    )

    bufs = (cbuf_a, cbuf_b)

    def slot(c, rows):
        b = bufs[c % 2]
        return b if rows == CROWS else b.at[pl.ds(0, rows)]

    reads = [None] * NCH
    writes = [None] * NCH
    for c in range(2):
        reads[c] = pltpu.async_copy(
            text_hbm.at[pl.ds(r0 + _OFFS[c], _SIZES[c])],
            slot(c, _SIZES[c]), rsem)
    for c in range(NCH):
        reads[c].wait()
        writes[c] = pltpu.async_copy(
            slot(c, _SIZES[c]),
            out_hbm.at[pl.ds(r0 + _OFFS[c], _SIZES[c])], wsem)
        n = c + 2
        if n < NCH:
            writes[c].wait()
            reads[n] = pltpu.async_copy(
                text_hbm.at[pl.ds(r0 + _OFFS[n], _SIZES[n])],
                slot(n, _SIZES[n]), rsem)
    writes[NCH - 2].wait()
    writes[NCH - 1].wait()

    def win_body(k, carry):
        base = jnp.minimum(lo8 + k * W, jnp.int32(NUM_TOKENS - W))
        base = pl.multiple_of(base, 8)
        ci = pltpu.async_copy(idx_hbm.at[pl.ds(base, W)], widx_v, isem)
        cs = pltpu.async_copy(src_hbm.at[pl.ds(base, W)], wsrc_v, isem)
        ci.wait()
        cs.wait()
        stage = cbuf_a.at[pl.ds(0, W)]
        pltpu.async_copy(image_hbm.at[wsrc_v], stage, gsem).wait()
        pltpu.async_copy(stage, out_hbm.at[widx_v], ssem).wait()
        return carry

    lax.fori_loop(0, nwin, win_body, jnp.int32(0))


@functools.cache
def _get_interleave():
    return pl.kernel(
        _body,
        out_type=jax.ShapeDtypeStruct((NUM_ROWS, HIDDEN), jnp.float32),
        mesh=plsc.VectorSubcoreMesh(
            core_axis_name="c",
            subcore_axis_name="s",
            num_cores=NUM_CORES,
            num_subcores=NUM_SUBCORES,
        ),
        scratch_types=[
            pltpu.VMEM((16,), jnp.int32),
            pltpu.VMEM((W,), jnp.int32),
            pltpu.VMEM((W,), jnp.int32),
            pltpu.VMEM((CROWS, HIDDEN), jnp.float32),
            pltpu.VMEM((CROWS, HIDDEN), jnp.float32),
            pltpu.SemaphoreType.DMA,
            pltpu.SemaphoreType.DMA,
            pltpu.SemaphoreType.DMA,
            pltpu.SemaphoreType.DMA,
            pltpu.SemaphoreType.DMA,
        ],
    )


def kernel(image_embeddings, text_embeddings, vision_indices):
    batch, seq_len, hidden = text_embeddings.shape
    flat = jnp.reshape(text_embeddings, (batch * seq_len, hidden))
    idx = vision_indices.astype(jnp.int32)
    iota = jnp.arange(NUM_TOKENS, dtype=jnp.int32)
    nxt = jnp.concatenate([idx[1:], jnp.full((1,), -1, jnp.int32)])
    src = lax.cummin(
        jnp.where(idx != nxt, iota, jnp.int32(NUM_TOKENS)), axis=0, reverse=True
    ).astype(jnp.int32)
    bounds = (jnp.int32(ROWS_PER_WORKER)
              * jnp.arange(NUM_WORKERS + 1, dtype=jnp.int32))[:, None]
    lo_ext = jnp.sum(idx[None, :] < bounds, axis=1, dtype=jnp.int32)
    lo_pad = jnp.concatenate([lo_ext, jnp.zeros((15,), jnp.int32)])
    lo_rows = jnp.stack(
        [lax.slice(lo_pad, (w,), (w + 16,)) for w in range(NUM_WORKERS)]
    )
    out = _get_interleave()(flat, image_embeddings, idx, src, lo_rows)
    return jnp.reshape(out, (batch, seq_len, hidden))


# speedup vs baseline: 1.2162x; 1.2162x over previous
import functools

import jax
import jax.numpy as jnp
from jax import lax
from jax.experimental import pallas as pl
from jax.experimental.pallas import tpu as pltpu
from jax.experimental.pallas import tpu_sc as plsc

HIDDEN = 2048
NUM_TOKENS = 4096
NUM_ROWS = 16384

NUM_CORES = 2
NUM_SUBCORES = 16
NUM_WORKERS = NUM_CORES * NUM_SUBCORES
PER_WORKER = NUM_TOKENS // NUM_WORKERS
CHUNK = 16
NUM_CHUNKS = PER_WORKER // CHUNK


NBUF = 3


def _scatter_body(image_hbm, idx_hbm, src_hbm, out_hbm, idx_v, src_v,
                  rows_a, rows_b, rows_c, gsem, ssem):
    wid = lax.axis_index("s") * NUM_CORES + lax.axis_index("c")
    pltpu.sync_copy(idx_hbm.at[wid], idx_v)
    pltpu.sync_copy(src_hbm.at[wid], src_v)
    bufs = (rows_a, rows_b, rows_c)
    gathers = [None] * NUM_CHUNKS
    scatters = [None] * NUM_CHUNKS
    for c in range(min(NBUF, NUM_CHUNKS)):
        gathers[c] = pltpu.async_copy(
            image_hbm.at[src_v.at[c]], bufs[c % NBUF], gsem)
    for c in range(NUM_CHUNKS):
        gathers[c].wait()
        scatters[c] = pltpu.async_copy(
            bufs[c % NBUF], out_hbm.at[idx_v.at[c]], ssem)
        n = c + NBUF
        if n < NUM_CHUNKS:
            scatters[c].wait()
            gathers[n] = pltpu.async_copy(
                image_hbm.at[src_v.at[n]], bufs[n % NBUF], gsem)
    for c in range(max(NUM_CHUNKS - NBUF, 0), NUM_CHUNKS):
        scatters[c].wait()


@functools.cache
def _get_scatter():
    return pl.kernel(
        _scatter_body,
        out_type=(),
        mesh=plsc.VectorSubcoreMesh(
            core_axis_name="c",
            subcore_axis_name="s",
            num_cores=NUM_CORES,
            num_subcores=NUM_SUBCORES,
        ),
        scratch_types=[
            pltpu.VMEM((NUM_CHUNKS, CHUNK), jnp.int32),
            pltpu.VMEM((NUM_CHUNKS, CHUNK), jnp.int32),
            pltpu.VMEM((CHUNK, HIDDEN), jnp.float32),
            pltpu.VMEM((CHUNK, HIDDEN), jnp.float32),
            pltpu.VMEM((CHUNK, HIDDEN), jnp.float32),
            pltpu.SemaphoreType.DMA,
            pltpu.SemaphoreType.DMA,
        ],
    )


def kernel(image_embeddings, text_embeddings, vision_indices):
    batch, seq_len, hidden = text_embeddings.shape
    flat = jnp.reshape(text_embeddings, (batch * seq_len, hidden))
    idx = vision_indices.astype(jnp.int32)
    iota = jnp.arange(NUM_TOKENS, dtype=jnp.int32)
    nxt = jnp.concatenate([idx[1:], jnp.full((1,), -1, jnp.int32)])
    src = lax.cummin(
        jnp.where(idx != nxt, iota, jnp.int32(NUM_TOKENS)), axis=0, reverse=True
    ).astype(jnp.int32)
    idx3 = jnp.reshape(idx, (NUM_WORKERS, NUM_CHUNKS, CHUNK))
    src3 = jnp.reshape(src, (NUM_WORKERS, NUM_CHUNKS, CHUNK))
    out_ref = jax.new_ref(flat)
    _get_scatter()(image_embeddings, idx3, src3, out_ref)
    return jnp.reshape(out_ref[...], (batch, seq_len, hidden))
